# Initial kernel scaffold; baseline (speedup 1.0000x reference)
import jax
import jax.numpy as jnp
from jax.experimental import pallas as pl


def kernel(modal0, modal1, modal2, fn_W1, fn_b1, fn_W2, fn_b2, g0_Wl, g0_bl, g0_Wr, g0_br, g0_att, g0_bias, ln0_g, ln0_b, g1_Wl, g1_bl, g1_Wr, g1_br, g1_att, g1_bias, ln1_g, ln1_b, conv_W, bn_g, bn_b):
    raise NotImplementedError("write your pallas kernel here")


# speedup vs baseline: 601.8635x; 601.8635x over previous
import functools

import jax
import jax.numpy as jnp
from jax.experimental import pallas as pl

C = 128
HEADS = 4
DH = C // HEADS
NN = 4


def _ln(o, g, b):
    mu = jnp.mean(o, axis=-1, keepdims=True)
    var = jnp.mean((o - mu) * (o - mu), axis=-1, keepdims=True)
    return (o - mu) / jnp.sqrt(var + 1e-5) * g + b


def _mm(a, b):
    return jnp.dot(a, b, preferred_element_type=jnp.float32)


def _gat_layer(Xlo, Xhi, Wl, bl, Wr, br, AG, bias, lg, lb):
    XLlo = _mm(Xlo, Wl) + bl
    XLhi = _mm(Xhi, Wl) + bl
    XRhi = _mm(Xhi, Wr) + br
    nlo = _ln(Xlo + XLlo + bias, lg, lb)
    s1 = XLlo + XRhi
    s1 = jnp.where(s1 >= 0, s1, 0.2 * s1)
    L1 = _mm(s1, AG)
    ss = XLhi + XRhi
    ss = jnp.where(ss >= 0, ss, 0.2 * ss)
    Ls = _mm(ss, AG)
    mx = jnp.maximum(L1, Ls)
    p1 = jnp.exp(L1 - mx)
    ps = jnp.exp(Ls - mx)
    out_hi = (6.0 * p1 * XLlo + ps * XLhi) / (6.0 * p1 + ps + 1e-16) + bias
    nhi = _ln(Xhi + out_hi, lg, lb)
    return nlo, nhi


def _gnn_body(l0, l1, l2, h0, h1, h2, fnW1, fnb1, fnW2, fnb2,
              Wl0, bl0, Wr0, br0, AG0, bias0, lg0, lb0,
              Wl1, bl1, Wr1, br1, AG1, bias1, lg1, lb1, outlo, outhi):
    T = l0.shape[2]
    alo = [l0[0].T, l1[0].T, l2[0].T]
    ahi = [h0[0].T, h1[0].T, h2[0].T]
    mean2 = jnp.concatenate([(alo[0] + alo[1] + alo[2]) * (1.0 / 3.0),
                             (ahi[0] + ahi[1] + ahi[2]) * (1.0 / 3.0)], axis=0)
    hmid = jnp.maximum(_mm(mean2, fnW1[...]) + fnb1[...], 0.0)
    fus2 = _mm(hmid, fnW2[...]) + fnb2[...]
    Xlo = jnp.concatenate([fus2[0:T]] + alo, axis=0)
    Xhi = jnp.concatenate([fus2[T:2 * T]] + ahi, axis=0)
    Xlo, Xhi = _gat_layer(Xlo, Xhi, Wl0[...], bl0[...], Wr0[...], br0[...],
                          AG0[...], bias0[...], lg0[...], lb0[...])
    Xlo, Xhi = _gat_layer(Xlo, Xhi, Wl1[...], bl1[...], Wr1[...], br1[...],
                          AG1[...], bias1[...], lg1[...], lb1[...])
    outlo[0] = Xlo[0:T]
    outhi[0] = Xhi[0:T]


def _conv_body(fr, wr, yr, statr, *, Wim):
    f2 = fr[0]
    HWn = f2.shape[0]
    z = jnp.zeros((Wim + 1, C), jnp.float32)
    fp = jnp.concatenate([z, f2, z], axis=0)
    wq = jax.lax.broadcasted_iota(jnp.int32, (HWn, 1), 0) % Wim
    acc = jnp.zeros((HWn, C), jnp.float32)
    for kh in range(3):
        for kw in range(3):
            off = Wim * (kh - 1) + (kw - 1)
            sl = jax.lax.slice(fp, (Wim + 1 + off, 0), (Wim + 1 + off + HWn, C))
            if kw == 0:
                sl = jnp.where(wq == 0, 0.0, sl)
            elif kw == 2:
                sl = jnp.where(wq == Wim - 1, 0.0, sl)
            acc = acc + _mm(sl, wr[3 * kh + kw])
    yr[0] = acc
    csum = jnp.sum(acc, axis=0, keepdims=True)
    csq = jnp.sum(acc * acc, axis=0, keepdims=True)
    statr[0] = jnp.concatenate([csum, csq, jnp.zeros((6, C), jnp.float32)], 0)


def _fin_body(yr, fr, statr, gr, br, outr, *, HW):
    total = jnp.sum(statr[:, 0:1, :], axis=0)
    totsq = jnp.sum(statr[:, 1:2, :], axis=0)
    cnt = jnp.float32(statr.shape[0] * HW)
    mu = total / cnt
    var = totsq / cnt - mu * mu
    rstd = 1.0 / jnp.sqrt(var + 1e-5)
    y = yr[0]
    yn = (y - mu) * rstd * gr[...] + br[...]
    o = jnp.maximum(yn, 0.0) + fr[0]
    outr[0] = o.T


def kernel(modal0, modal1, modal2, fn_W1, fn_b1, fn_W2, fn_b2,
           g0_Wl, g0_bl, g0_Wr, g0_br, g0_att, g0_bias, ln0_g, ln0_b,
           g1_Wl, g1_bl, g1_Wr, g1_br, g1_att, g1_bias, ln1_g, ln1_b,
           conv_W, bn_g, bn_b):
    B, Cc, H, W = modal0.shape
    HW = H * W
    Bh = B // 2
    T = min(1024, HW)
    m0 = modal0.reshape(B, Cc, HW)
    m1 = modal1.reshape(B, Cc, HW)
    m2 = modal2.reshape(B, Cc, HW)

    gid = jnp.arange(C) // DH
    gmask = (gid[:, None] == gid[None, :]).astype(jnp.float32)
    AG0 = g0_att.reshape(C)[:, None] * gmask
    AG1 = g1_att.reshape(C)[:, None] * gmask

    row = lambda v: v.reshape(1, C)
    wfull = lambda: pl.BlockSpec((C, C), lambda b, t: (0, 0))
    rfull = lambda: pl.BlockSpec((1, C), lambda b, t: (0, 0))
    mlo = pl.BlockSpec((1, Cc, T), lambda b, t: (b, 0, t))
    mhi = pl.BlockSpec((1, Cc, T), lambda b, t: (b + Bh, 0, t))

    flo, fhi = pl.pallas_call(
        _gnn_body,
        grid=(Bh, HW // T),
        in_specs=[mlo, mlo, mlo, mhi, mhi, mhi,
                  wfull(), rfull(), wfull(), rfull(),
                  wfull(), rfull(), wfull(), rfull(), wfull(), rfull(), rfull(), rfull(),
                  wfull(), rfull(), wfull(), rfull(), wfull(), rfull(), rfull(), rfull()],
        out_specs=[pl.BlockSpec((1, T, C), lambda b, t: (b, t, 0)),
                   pl.BlockSpec((1, T, C), lambda b, t: (b, t, 0))],
        out_shape=[jax.ShapeDtypeStruct((Bh, HW, C), jnp.float32),
                   jax.ShapeDtypeStruct((Bh, HW, C), jnp.float32)],
    )(m0, m1, m2, m0, m1, m2,
      fn_W1, row(fn_b1), fn_W2, row(fn_b2),
      g0_Wl, row(g0_bl), g0_Wr, row(g0_br), AG0, row(g0_bias), row(ln0_g), row(ln0_b),
      g1_Wl, row(g1_bl), g1_Wr, row(g1_br), AG1, row(g1_bias), row(ln1_g), row(ln1_b))

    fused = jnp.concatenate([flo, fhi], axis=0)

    Wc = jnp.transpose(conv_W, (2, 3, 1, 0)).reshape(9, C, C)

    y, stats = pl.pallas_call(
        functools.partial(_conv_body, Wim=W),
        grid=(B,),
        in_specs=[pl.BlockSpec((1, HW, C), lambda b: (b, 0, 0)),
                  pl.BlockSpec((9, C, C), lambda b: (0, 0, 0))],
        out_specs=[pl.BlockSpec((1, HW, C), lambda b: (b, 0, 0)),
                   pl.BlockSpec((1, 8, C), lambda b: (b, 0, 0))],
        out_shape=[jax.ShapeDtypeStruct((B, HW, C), jnp.float32),
                   jax.ShapeDtypeStruct((B, 8, C), jnp.float32)],
    )(fused, Wc)

    T2 = min(1024, HW)
    out = pl.pallas_call(
        functools.partial(_fin_body, HW=HW),
        grid=(B, HW // T2),
        in_specs=[pl.BlockSpec((1, T2, C), lambda b, t: (b, t, 0)),
                  pl.BlockSpec((1, T2, C), lambda b, t: (b, t, 0)),
                  pl.BlockSpec((B, 8, C), lambda b, t: (0, 0, 0)),
                  rfull(), rfull()],
        out_specs=pl.BlockSpec((1, C, T2), lambda b, t: (b, 0, t)),
        out_shape=jax.ShapeDtypeStruct((B, C, HW), jnp.float32),
    )(y, fused, stats, row(bn_g), row(bn_b))

    return out.reshape(B, C, H, W)


# speedup vs baseline: 661.5774x; 1.0992x over previous
import functools

import jax
import jax.numpy as jnp
from jax.experimental import pallas as pl

C = 128
HEADS = 4
DH = C // HEADS
NN = 4


def _ln(o, g, b):
    mu = jnp.mean(o, axis=-1, keepdims=True)
    var = jnp.mean((o - mu) * (o - mu), axis=-1, keepdims=True)
    return (o - mu) * jax.lax.rsqrt(var + 1e-5) * g + b


def _mm(a, b):
    return jnp.dot(a, b, preferred_element_type=jnp.float32)


def _gat_layer(Xlo, Xhi, Wl, bl, Wr, br, AG, bias, lg, lb):
    n = Xlo.shape[0]
    XL2 = _mm(jnp.concatenate([Xlo, Xhi], axis=0), Wl) + bl
    XLlo = XL2[0:n]
    XLhi = XL2[n:2 * n]
    XRhi = _mm(Xhi, Wr) + br
    nlo = _ln(Xlo + XLlo + bias, lg, lb)
    s1 = XLlo + XRhi
    s1 = jnp.maximum(s1, 0.2 * s1)
    ss = XLhi + XRhi
    ss = jnp.maximum(ss, 0.2 * ss)
    d = _mm(s1 - ss, AG)
    a1 = 1.0 / (1.0 + jnp.exp(-d) * (1.0 / 6.0))
    out_hi = XLhi + a1 * (XLlo - XLhi) + bias
    nhi = _ln(Xhi + out_hi, lg, lb)
    return nlo, nhi


def _gnn_body(l0, l1, l2, h0, h1, h2, fnW1, fnb1, fnW2, fnb2,
              Wl0, bl0, Wr0, br0, AG0, bias0, lg0, lb0,
              Wl1, bl1, Wr1, br1, AG1, bias1, lg1, lb1, outlo, outhi):
    T = l0.shape[2]
    alo = [l0[0].T, l1[0].T, l2[0].T]
    ahi = [h0[0].T, h1[0].T, h2[0].T]
    mean2 = jnp.concatenate([(alo[0] + alo[1] + alo[2]) * (1.0 / 3.0),
                             (ahi[0] + ahi[1] + ahi[2]) * (1.0 / 3.0)], axis=0)
    hmid = jnp.maximum(_mm(mean2, fnW1[...]) + fnb1[...], 0.0)
    fus2 = _mm(hmid, fnW2[...]) + fnb2[...]
    Xlo = jnp.concatenate([fus2[0:T]] + alo, axis=0)
    Xhi = jnp.concatenate([fus2[T:2 * T]] + ahi, axis=0)
    Xlo, Xhi = _gat_layer(Xlo, Xhi, Wl0[...], bl0[...], Wr0[...], br0[...],
                          AG0[...], bias0[...], lg0[...], lb0[...])
    Xlo, Xhi = _gat_layer(Xlo, Xhi, Wl1[...], bl1[...], Wr1[...], br1[...],
                          AG1[...], bias1[...], lg1[...], lb1[...])
    outlo[0] = Xlo[0:T]
    outhi[0] = Xhi[0:T]


def _conv_body(fr, wr, yr, statr, *, Wim):
    f2 = fr[0]
    HWn = f2.shape[0]
    z = jnp.zeros((Wim + 1, C), jnp.float32)
    fp = jnp.concatenate([z, f2, z], axis=0)
    wq = jax.lax.broadcasted_iota(jnp.int32, (HWn, 1), 0) % Wim
    acc = jnp.zeros((HWn, C), jnp.float32)
    for kh in range(3):
        for kw in range(3):
            off = Wim * (kh - 1) + (kw - 1)
            sl = jax.lax.slice(fp, (Wim + 1 + off, 0), (Wim + 1 + off + HWn, C))
            if kw == 0:
                sl = jnp.where(wq == 0, 0.0, sl)
            elif kw == 2:
                sl = jnp.where(wq == Wim - 1, 0.0, sl)
            acc = acc + _mm(sl, wr[3 * kh + kw])
    yr[0] = acc
    csum = jnp.sum(acc, axis=0, keepdims=True)
    csq = jnp.sum(acc * acc, axis=0, keepdims=True)
    statr[0] = jnp.concatenate([csum, csq, jnp.zeros((6, C), jnp.float32)], 0)


def _fin_body(yr, fr, statr, gr, br, outr, *, HW):
    total = jnp.sum(statr[:, 0:1, :], axis=0)
    totsq = jnp.sum(statr[:, 1:2, :], axis=0)
    cnt = jnp.float32(statr.shape[0] * HW)
    mu = total / cnt
    var = totsq / cnt - mu * mu
    rstd = 1.0 / jnp.sqrt(var + 1e-5)
    y = yr[0]
    yn = (y - mu) * rstd * gr[...] + br[...]
    o = jnp.maximum(yn, 0.0) + fr[0]
    outr[0] = o.T


def kernel(modal0, modal1, modal2, fn_W1, fn_b1, fn_W2, fn_b2,
           g0_Wl, g0_bl, g0_Wr, g0_br, g0_att, g0_bias, ln0_g, ln0_b,
           g1_Wl, g1_bl, g1_Wr, g1_br, g1_att, g1_bias, ln1_g, ln1_b,
           conv_W, bn_g, bn_b):
    B, Cc, H, W = modal0.shape
    HW = H * W
    Bh = B // 2
    T = min(1024, HW)
    m0 = modal0.reshape(B, Cc, HW)
    m1 = modal1.reshape(B, Cc, HW)
    m2 = modal2.reshape(B, Cc, HW)

    gid = jnp.arange(C) // DH
    gmask = (gid[:, None] == gid[None, :]).astype(jnp.float32)
    AG0 = g0_att.reshape(C)[:, None] * gmask
    AG1 = g1_att.reshape(C)[:, None] * gmask

    row = lambda v: v.reshape(1, C)
    wfull = lambda: pl.BlockSpec((C, C), lambda b, t: (0, 0))
    rfull = lambda: pl.BlockSpec((1, C), lambda b, t: (0, 0))
    mlo = pl.BlockSpec((1, Cc, T), lambda b, t: (b, 0, t))
    mhi = pl.BlockSpec((1, Cc, T), lambda b, t: (b + Bh, 0, t))

    flo, fhi = pl.pallas_call(
        _gnn_body,
        grid=(Bh, HW // T),
        in_specs=[mlo, mlo, mlo, mhi, mhi, mhi,
                  wfull(), rfull(), wfull(), rfull(),
                  wfull(), rfull(), wfull(), rfull(), wfull(), rfull(), rfull(), rfull(),
                  wfull(), rfull(), wfull(), rfull(), wfull(), rfull(), rfull(), rfull()],
        out_specs=[pl.BlockSpec((1, T, C), lambda b, t: (b, t, 0)),
                   pl.BlockSpec((1, T, C), lambda b, t: (b, t, 0))],
        out_shape=[jax.ShapeDtypeStruct((Bh, HW, C), jnp.float32),
                   jax.ShapeDtypeStruct((Bh, HW, C), jnp.float32)],
    )(m0, m1, m2, m0, m1, m2,
      fn_W1, row(fn_b1), fn_W2, row(fn_b2),
      g0_Wl, row(g0_bl), g0_Wr, row(g0_br), AG0, row(g0_bias), row(ln0_g), row(ln0_b),
      g1_Wl, row(g1_bl), g1_Wr, row(g1_br), AG1, row(g1_bias), row(ln1_g), row(ln1_b))

    fused = jnp.concatenate([flo, fhi], axis=0)

    Wc = jnp.transpose(conv_W, (2, 3, 1, 0)).reshape(9, C, C)

    y, stats = pl.pallas_call(
        functools.partial(_conv_body, Wim=W),
        grid=(B,),
        in_specs=[pl.BlockSpec((1, HW, C), lambda b: (b, 0, 0)),
                  pl.BlockSpec((9, C, C), lambda b: (0, 0, 0))],
        out_specs=[pl.BlockSpec((1, HW, C), lambda b: (b, 0, 0)),
                   pl.BlockSpec((1, 8, C), lambda b: (b, 0, 0))],
        out_shape=[jax.ShapeDtypeStruct((B, HW, C), jnp.float32),
                   jax.ShapeDtypeStruct((B, 8, C), jnp.float32)],
    )(fused, Wc)

    T2 = min(1024, HW)
    out = pl.pallas_call(
        functools.partial(_fin_body, HW=HW),
        grid=(B, HW // T2),
        in_specs=[pl.BlockSpec((1, T2, C), lambda b, t: (b, t, 0)),
                  pl.BlockSpec((1, T2, C), lambda b, t: (b, t, 0)),
                  pl.BlockSpec((B, 8, C), lambda b, t: (0, 0, 0)),
                  rfull(), rfull()],
        out_specs=pl.BlockSpec((1, C, T2), lambda b, t: (b, 0, t)),
        out_shape=jax.ShapeDtypeStruct((B, C, HW), jnp.float32),
    )(y, fused, stats, row(bn_g), row(bn_b))

    return out.reshape(B, C, H, W)
